# baseline (speedup 1.0000x reference)
import functools

import jax
import jax.numpy as jnp
from jax import lax
from jax.experimental import pallas as pl
from jax.experimental.pallas import tpu as pltpu
from jax.experimental.pallas import tpu_sc as plsc

BATCH = 16384
DIM = 32
NFEAT = 26
NW = 32
BPW = BATCH // NW
IDX_W = 128
ROWS_PER_W = BPW // IDX_W
NPARAM = 80
QSCALE = 4096.0


def _sc_body(uix_hbm, mix_hbm, feat_hbm, utab_hbm, mtab_hbm, par_hbm,
             out_hbm, uix_v, mix_v, u_v, m_v, f_v, p_v, o_v, sem):
    nc = 2
    wid = lax.axis_index("s") * nc + lax.axis_index("c")
    base = wid * BPW

    pltpu.sync_copy(uix_hbm.at[pl.ds(base, BPW)], uix_v)
    pltpu.sync_copy(mix_hbm.at[pl.ds(base, BPW)], mix_v)
    pltpu.sync_copy(par_hbm, p_v)

    copies = [pltpu.async_copy(feat_hbm.at[pl.ds(base, BPW)], f_v, sem)]
    for j in range(ROWS_PER_W):
        copies.append(pltpu.async_copy(
            utab_hbm.at[uix_v.at[pl.ds(j * IDX_W, IDX_W)]],
            u_v.at[pl.ds(j * IDX_W, IDX_W)], sem))
        copies.append(pltpu.async_copy(
            mtab_hbm.at[mix_v.at[pl.ds(j * IDX_W, IDX_W)]],
            m_v.at[pl.ds(j * IDX_W, IDX_W)], sem))
    for c in copies:
        c.wait()

    we = p_v[pl.ds(0, 16)]
    wo = p_v[pl.ds(16, 16)]
    wf0 = p_v[pl.ds(32, 16)]
    wf1 = p_v[pl.ds(48, 16)]
    b_vec = p_v[pl.ds(64, 16)]
    lane = lax.iota(jnp.int32, 16)

    def lanesum(t):
        for k in (1, 2, 4, 8):
            t = t + t.at[lane ^ k].get(mode="promise_in_bounds",
                                       unique_indices=True)
        return t

    def group(g, carry):
        base_i = g * 16
        acc = jnp.zeros((16,), jnp.float32)
        for j in range(16):
            i = base_i + j
            ui = u_v[i, pl.ds(0, 16)]
            mi = m_v[i, pl.ds(0, 16)]
            ue = ((ui << 16) >> 16).astype(jnp.float32)
            uo = (ui >> 16).astype(jnp.float32)
            me = ((mi << 16) >> 16).astype(jnp.float32)
            mo = (mi >> 16).astype(jnp.float32)
            f0 = f_v[i, pl.ds(0, 16)]
            f1 = f_v[i, pl.ds(16, 16)]
            t = (ue * me) * we + (uo * mo) * wo + f0 * wf0 + f1 * wf1
            acc = jnp.where(lane == j, lanesum(t), acc)
        o_v[pl.ds(base_i, 16)] = acc + b_vec
        return carry

    lax.fori_loop(0, BPW // 16, group, 0)

    pltpu.sync_copy(o_v, out_hbm.at[pl.ds(base, BPW)])


_sc_call = functools.partial(
    pl.kernel,
    mesh=plsc.VectorSubcoreMesh(core_axis_name="c", subcore_axis_name="s"),
    out_type=jax.ShapeDtypeStruct((BATCH,), jnp.float32),
    compiler_params=pltpu.CompilerParams(use_tc_tiling_on_sc=False),
    scratch_types=[
        pltpu.VMEM((BPW,), jnp.int32),
        pltpu.VMEM((BPW,), jnp.int32),
        pltpu.VMEM((BPW, DIM // 2), jnp.int32),
        pltpu.VMEM((BPW, DIM // 2), jnp.int32),
        pltpu.VMEM((BPW, DIM), jnp.float32),
        pltpu.VMEM((NPARAM,), jnp.float32),
        pltpu.VMEM((BPW,), jnp.float32),
        pltpu.SemaphoreType.DMA,
    ],
)(_sc_body)


def kernel(user_idx, movie_idx, features, user_table, movie_table, fc_w, fc_b):
    uix = user_idx.astype(jnp.int32)
    mix = movie_idx.astype(jnp.int32)
    featp = jnp.pad(features, ((0, 0), (0, DIM - NFEAT)))
    w = fc_w.reshape(-1)
    inv_sq = jnp.float32(1.0 / (QSCALE * QSCALE))
    params = jnp.concatenate([
        w[:DIM:2] * inv_sq,
        w[1:DIM:2] * inv_sq,
        jnp.pad(w[DIM:], (0, DIM - NFEAT)),
        jnp.broadcast_to(fc_b.reshape(-1), (16,)),
    ])
    def pack16(t):
        q = jnp.clip(jnp.round(t * QSCALE), -32767.0, 32767.0)
        q = q.astype(jnp.int16).reshape(t.shape[0], DIM // 2, 2)
        return jax.lax.bitcast_convert_type(q, jnp.int32)
    return _sc_call(uix, mix, featp, pack16(user_table),
                    pack16(movie_table), params)


# speedup vs baseline: 2.1026x; 2.1026x over previous
import functools

import jax
import jax.numpy as jnp
from jax import lax
from jax.experimental import pallas as pl
from jax.experimental.pallas import tpu as pltpu
from jax.experimental.pallas import tpu_sc as plsc

BATCH = 16384
DIM = 32
NFEAT = 26
NW = 32
BPW = BATCH // NW
CHUNK = 128
NCHUNK = BPW // CHUNK
FROWS = BATCH * DIM // 128


def _sc_body(uix_hbm, mix_hbm, feat_hbm, wu_hbm, wm_hbm, par_hbm, out_hbm,
             rawu, rawm, giu, gim, u_c, m_c, f_c, p_v, o_v, sem):
    wid = lax.axis_index("c") * 16 + lax.axis_index("s")
    base = wid * BPW
    lane = lax.iota(jnp.int32, 16)

    pltpu.sync_copy(par_hbm, p_v)

    for c4 in range(NCHUNK):
        pltpu.sync_copy(uix_hbm.at[pl.ds(base + c4 * CHUNK, CHUNK)], rawu)
        pltpu.sync_copy(mix_hbm.at[pl.ds(base + c4 * CHUNK, CHUNK)], rawm)

        def prep(g, carry):
            ru = rawu[pl.ds(g * 16, 16)]
            rm = rawm[pl.ds(g * 16, 16)]
            giu[pl.ds(g * 16, 16)] = ru >> 2
            gim[pl.ds(g * 16, 16)] = rm >> 2
            return carry

        lax.fori_loop(0, CHUNK // 16, prep, 0)

        cu = pltpu.async_copy(wu_hbm.at[giu], u_c, sem)
        cm = pltpu.async_copy(wm_hbm.at[gim], m_c, sem)
        cf = pltpu.async_copy(
            feat_hbm.at[pl.ds(wid * 128 + c4 * 32, 32)], f_c, sem)
        cu.wait()
        cm.wait()
        cf.wait()

        def group(g, carry):
            ru = rawu[pl.ds(g * 16, 16)]
            rm = rawm[pl.ds(g * 16, 16)]
            qu = (ru & 3) * DIM
            qm = (rm & 3) * DIM
            jloc = lane + g * 16
            jw = jloc * DIM
            acc = p_v[pl.ds((DIM + NFEAT) * 16, 16)]
            for d in range(DIM):
                uv = plsc.load_gather(u_c, [jloc, qu + d])
                mv = plsc.load_gather(m_c, [jloc, qm + d])
                wd = p_v[pl.ds(d * 16, 16)]
                acc = acc + uv * mv * wd
            for d in range(NFEAT):
                off = jw + d
                fv = plsc.load_gather(f_c, [off >> 7, off & 127])
                wd = p_v[pl.ds((DIM + d) * 16, 16)]
                acc = acc + fv * wd
            o_v[pl.ds(c4 * CHUNK + g * 16, 16)] = acc
            return carry

        lax.fori_loop(0, CHUNK // 16, group, 0)

    pltpu.sync_copy(o_v, out_hbm.at[pl.ds(base, BPW)])


_sc_call = functools.partial(
    pl.kernel,
    mesh=plsc.VectorSubcoreMesh(core_axis_name="c", subcore_axis_name="s"),
    out_type=jax.ShapeDtypeStruct((BATCH,), jnp.float32),
    compiler_params=pltpu.CompilerParams(needs_layout_passes=False),
    scratch_types=[
        pltpu.VMEM((CHUNK,), jnp.int32),
        pltpu.VMEM((CHUNK,), jnp.int32),
        pltpu.VMEM((CHUNK,), jnp.int32),
        pltpu.VMEM((CHUNK,), jnp.int32),
        pltpu.VMEM((CHUNK, 128), jnp.float32),
        pltpu.VMEM((CHUNK, 128), jnp.float32),
        pltpu.VMEM((DIM, 128), jnp.float32),
        pltpu.VMEM(((DIM + NFEAT + 1) * 16,), jnp.float32),
        pltpu.VMEM((BPW,), jnp.float32),
        pltpu.SemaphoreType.DMA,
    ],
)(_sc_body)


def kernel(user_idx, movie_idx, features, user_table, movie_table, fc_w, fc_b):
    uix = user_idx.astype(jnp.int32)
    mix = movie_idx.astype(jnp.int32)
    nu = user_table.shape[0] - 1
    nm = movie_table.shape[0] - 1
    wu = user_table[:nu].reshape(nu // 4, 128)
    wm = movie_table[:nm].reshape(nm // 4, 128)
    featw = jnp.pad(features, ((0, 0), (0, DIM - NFEAT))).reshape(FROWS, 128)
    wb = jnp.concatenate([fc_w.reshape(-1), fc_b.reshape(-1)])
    params = jnp.repeat(wb, 16)
    return _sc_call(uix, mix, featw, wu, wm, params)


# speedup vs baseline: 2.1156x; 1.0062x over previous
import functools

import jax
import jax.numpy as jnp
from jax import lax
from jax.experimental import pallas as pl
from jax.experimental.pallas import tpu as pltpu
from jax.experimental.pallas import tpu_sc as plsc

BATCH = 16384
DIM = 32
NFEAT = 26
NW = 32
BPW = BATCH // NW
CHUNK = 128
NCHUNK = BPW // CHUNK
FROWS = BATCH * DIM // 128


def _sc_body(uix_hbm, mix_hbm, feat_hbm, wu_hbm, wm_hbm, par_hbm, out_hbm,
             rawu, rawm, u_c, m_c, f_c, p_v, o_v, sem):
    wid = lax.axis_index("c") * 16 + lax.axis_index("s")
    base = wid * BPW
    lane = lax.iota(jnp.int32, 16)

    pltpu.sync_copy(par_hbm, p_v)

    for c4 in range(NCHUNK):
        pltpu.sync_copy(uix_hbm.at[pl.ds(base + c4 * CHUNK, CHUNK)], rawu)
        pltpu.sync_copy(mix_hbm.at[pl.ds(base + c4 * CHUNK, CHUNK)], rawm)

        cu = pltpu.async_copy(wu_hbm.at[rawu], u_c, sem)
        cm = pltpu.async_copy(wm_hbm.at[rawm], m_c, sem)
        cf = pltpu.async_copy(
            feat_hbm.at[pl.ds(wid * 128 + c4 * 32, 32)], f_c, sem)
        cu.wait()
        cm.wait()
        cf.wait()

        def group(g, carry):
            jloc = lane + g * 16
            jw = jloc * DIM
            dz = jnp.zeros((16,), jnp.int32)
            acc = p_v[pl.ds((DIM + NFEAT) * 16, 16)]
            for d in range(DIM):
                uv = plsc.load_gather(u_c, [jloc, dz + d])
                mv = plsc.load_gather(m_c, [jloc, dz + d])
                wd = p_v[pl.ds(d * 16, 16)]
                acc = acc + uv * mv * wd
            for d in range(NFEAT):
                off = jw + d
                fv = plsc.load_gather(f_c, [off >> 7, off & 127])
                wd = p_v[pl.ds((DIM + d) * 16, 16)]
                acc = acc + fv * wd
            o_v[pl.ds(c4 * CHUNK + g * 16, 16)] = acc
            return carry

        lax.fori_loop(0, CHUNK // 16, group, 0)

    pltpu.sync_copy(o_v, out_hbm.at[pl.ds(base, BPW)])


_sc_call = functools.partial(
    pl.kernel,
    mesh=plsc.VectorSubcoreMesh(core_axis_name="c", subcore_axis_name="s"),
    out_type=jax.ShapeDtypeStruct((BATCH,), jnp.float32),
    compiler_params=pltpu.CompilerParams(needs_layout_passes=False),
    scratch_types=[
        pltpu.VMEM((CHUNK,), jnp.int32),
        pltpu.VMEM((CHUNK,), jnp.int32),
        pltpu.VMEM((CHUNK, 128), jnp.float32),
        pltpu.VMEM((CHUNK, 128), jnp.float32),
        pltpu.VMEM((DIM, 128), jnp.float32),
        pltpu.VMEM(((DIM + NFEAT + 1) * 16,), jnp.float32),
        pltpu.VMEM((BPW,), jnp.float32),
        pltpu.SemaphoreType.DMA,
    ],
)(_sc_body)


def kernel(user_idx, movie_idx, features, user_table, movie_table, fc_w, fc_b):
    uix = user_idx.astype(jnp.int32)
    mix = movie_idx.astype(jnp.int32)
    wu = jnp.pad(user_table, ((0, 0), (0, 128 - DIM)))
    wm = jnp.pad(movie_table, ((0, 0), (0, 128 - DIM)))
    featw = jnp.pad(features, ((0, 0), (0, DIM - NFEAT))).reshape(FROWS, 128)
    wb = jnp.concatenate([fc_w.reshape(-1), fc_b.reshape(-1)])
    params = jnp.repeat(wb, 16)
    return _sc_call(uix, mix, featw, wu, wm, params)


# speedup vs baseline: 2.2238x; 1.0511x over previous
import functools

import jax
import jax.numpy as jnp
from jax import lax
from jax.experimental import pallas as pl
from jax.experimental.pallas import tpu as pltpu
from jax.experimental.pallas import tpu_sc as plsc

BATCH = 16384
DIM = 32
NFEAT = 26
NW = 32
BPW = BATCH // NW
IDX_W = 128
ROWS_PER_W = BPW // IDX_W
NPARAM = 80


def _sc_body(uix_hbm, mix_hbm, feat_hbm, utab_hbm, mtab_hbm, par_hbm,
             out_hbm, uix_v, mix_v, u_v, m_v, f_v, p_v, o_v, sem):
    nc = 2
    wid = lax.axis_index("s") * nc + lax.axis_index("c")
    base = wid * BPW

    pltpu.sync_copy(uix_hbm.at[pl.ds(base, BPW)], uix_v)
    pltpu.sync_copy(mix_hbm.at[pl.ds(base, BPW)], mix_v)
    pltpu.sync_copy(par_hbm, p_v)

    copies = [pltpu.async_copy(feat_hbm.at[pl.ds(base, BPW)], f_v, sem)]
    for j in range(ROWS_PER_W):
        copies.append(pltpu.async_copy(
            utab_hbm.at[uix_v.at[pl.ds(j * IDX_W, IDX_W)]],
            u_v.at[pl.ds(j * IDX_W, IDX_W)], sem))
        copies.append(pltpu.async_copy(
            mtab_hbm.at[mix_v.at[pl.ds(j * IDX_W, IDX_W)]],
            m_v.at[pl.ds(j * IDX_W, IDX_W)], sem))
    for c in copies:
        c.wait()

    w0 = p_v[pl.ds(0, 16)]
    w1 = p_v[pl.ds(16, 16)]
    wf0 = p_v[pl.ds(32, 16)]
    wf1 = p_v[pl.ds(48, 16)]
    b_vec = p_v[pl.ds(64, 16)]
    lane = lax.iota(jnp.int32, 16)

    def lanesum(t):
        for k in (1, 2, 4, 8):
            t = t + t.at[lane ^ k].get(mode="promise_in_bounds",
                                       unique_indices=True)
        return t

    def group(g, carry):
        base_i = g * 16
        acc = jnp.zeros((16,), jnp.float32)
        for j in range(16):
            i = base_i + j
            u0 = u_v[i, pl.ds(0, 16)]
            u1 = u_v[i, pl.ds(16, 16)]
            m0 = m_v[i, pl.ds(0, 16)]
            m1 = m_v[i, pl.ds(16, 16)]
            f0 = f_v[i, pl.ds(0, 16)]
            f1 = f_v[i, pl.ds(16, 16)]
            t = u0 * m0 * w0 + u1 * m1 * w1 + f0 * wf0 + f1 * wf1
            acc = jnp.where(lane == j, lanesum(t), acc)
        o_v[pl.ds(base_i, 16)] = acc + b_vec
        return carry

    lax.fori_loop(0, BPW // 16, group, 0)

    pltpu.sync_copy(o_v, out_hbm.at[pl.ds(base, BPW)])


_sc_call = functools.partial(
    pl.kernel,
    mesh=plsc.VectorSubcoreMesh(core_axis_name="c", subcore_axis_name="s"),
    out_type=jax.ShapeDtypeStruct((BATCH,), jnp.float32),
    compiler_params=pltpu.CompilerParams(use_tc_tiling_on_sc=False),
    scratch_types=[
        pltpu.VMEM((BPW,), jnp.int32),
        pltpu.VMEM((BPW,), jnp.int32),
        pltpu.VMEM((BPW, DIM), jnp.float32),
        pltpu.VMEM((BPW, DIM), jnp.float32),
        pltpu.VMEM((BPW, DIM), jnp.float32),
        pltpu.VMEM((NPARAM,), jnp.float32),
        pltpu.VMEM((BPW,), jnp.float32),
        pltpu.SemaphoreType.DMA,
    ],
)(_sc_body)


def kernel(user_idx, movie_idx, features, user_table, movie_table, fc_w, fc_b):
    uix = user_idx.astype(jnp.int32)
    mix = movie_idx.astype(jnp.int32)
    featp = jnp.pad(features, ((0, 0), (0, DIM - NFEAT)))
    w = fc_w.reshape(-1)
    params = jnp.concatenate([
        w[:DIM],
        jnp.pad(w[DIM:], (0, DIM - NFEAT)),
        jnp.broadcast_to(fc_b.reshape(-1), (16,)),
    ])
    return _sc_call(uix, mix, featp, user_table, movie_table, params)
